# Initial kernel scaffold; baseline (speedup 1.0000x reference)
import functools

import jax
import jax.numpy as jnp
from jax import lax
from jax.experimental import pallas as pl
from jax.experimental.pallas import tpu as pltpu
from jax.experimental.pallas import tpu_sc as plsc

N = 10000
E = 320000
D = 128
B = 64
OUT = 10

NC = 2
NS = 16
NW = NC * NS
EW = E // NW
C = 128
NCHUNK = -(-EW // C)
SLAB = NCHUNK * C
NACC = 10240
TRASH = N
RPT = NACC // NS
F32 = jnp.float32

_mesh = plsc.VectorSubcoreMesh(core_axis_name="c", subcore_axis_name="s")


def _zero_buf(buf, rows, width):
    z = jnp.zeros((16,), F32)

    def row(r, carry):
        for k in range(width // 16):
            buf[r, pl.ds(k * 16, 16)] = z
        return carry

    lax.fori_loop(0, rows, row, 0)


@functools.partial(
    pl.kernel,
    out_type=jax.ShapeDtypeStruct((NC, NACC, 128), F32),
    mesh=_mesh,
    scratch_types=[
        pltpu.VMEM((NCHUNK, C), jnp.int32),
        pltpu.VMEM((NCHUNK, C), jnp.int32),
        pltpu.VMEM((C, 128), F32),
        pltpu.VMEM_SHARED((NACC, 128), F32),
        pltpu.SemaphoreType.DMA,
    ],
)
def _edge_scatter(y, srch, dsth, out, srcv, dstv, buf, acc, sem):
    c = lax.axis_index("c")
    s = lax.axis_index("s")
    wid = c * NS + s

    pltpu.sync_copy(srch.at[pl.ds(wid * NCHUNK, NCHUNK)], srcv)
    pltpu.sync_copy(dsth.at[pl.ds(wid * NCHUNK, NCHUNK)], dstv)

    _zero_buf(buf, 128, 128)
    base = s * RPT
    for off in range(0, RPT, 128):
        pltpu.sync_copy(buf, acc.at[pl.ds(base + off, 128)])
    plsc.subcore_barrier()

    def step(j, carry):
        pltpu.async_copy(y.at[srcv.at[j]], buf, sem).wait()
        pltpu.sync_copy(buf, acc.at[dstv.at[j]], add=True)
        return carry

    lax.fori_loop(0, NCHUNK, step, 0)
    plsc.subcore_barrier()

    for off in range(0, RPT, 128):
        pltpu.sync_copy(acc.at[pl.ds(base + off, 128)], buf)
        pltpu.sync_copy(buf, out.at[c, pl.ds(base + off, 128)])


@functools.partial(
    pl.kernel,
    out_type=jax.ShapeDtypeStruct((NC, NACC, 128), F32),
    mesh=_mesh,
    scratch_types=[
        pltpu.VMEM((NCHUNK, C), jnp.int32),
        pltpu.VMEM((C, 16), F32),
        pltpu.VMEM((C, 16), F32),
        pltpu.VMEM((C, 128), F32),
        pltpu.VMEM_SHARED((NACC, 16), F32),
    ],
)
def _degree(dsth, out, dstv, ones, dbuf, bbuf, acc):
    c = lax.axis_index("c")
    s = lax.axis_index("s")
    wid = c * NS + s

    pltpu.sync_copy(dsth.at[pl.ds(wid * NCHUNK, NCHUNK)], dstv)

    one = jnp.ones((16,), F32)

    def fill(r, carry):
        ones[r, pl.ds(0, 16)] = one
        return carry

    lax.fori_loop(0, C, fill, 0)
    _zero_buf(dbuf, C, 16)

    base = s * RPT
    for off in range(0, RPT, 128):
        pltpu.sync_copy(dbuf, acc.at[pl.ds(base + off, 128)])
    plsc.subcore_barrier()

    def step(j, carry):
        pltpu.sync_copy(ones, acc.at[dstv.at[j]], add=True)
        return carry

    lax.fori_loop(0, NCHUNK, step, 0)
    plsc.subcore_barrier()

    for off in range(0, RPT, 128):
        pltpu.sync_copy(acc.at[pl.ds(base + off, 128)], dbuf)

        def brow(r, carry):
            v = dbuf[r, pl.ds(0, 16)]
            for k in range(8):
                bbuf[r, pl.ds(k * 16, 16)] = v
            return carry

        lax.fori_loop(0, C, brow, 0)
        pltpu.sync_copy(bbuf, out.at[c, pl.ds(base + off, 128)])


def _k1_body(x_ref, w_ref, da_ref, db_ref, y_ref, dinv_ref):
    deg = da_ref[...] + db_ref[...] + 1.0
    dinv = lax.rsqrt(deg)
    xw = jnp.dot(x_ref[...], w_ref[...], preferred_element_type=F32)
    y_ref[...] = dinv * xw
    dinv_ref[...] = dinv


def _k2_body(a0_ref, a1_ref, y_ref, dinv_ref, b_ref, w_ref, y2_ref):
    dinv = dinv_ref[...]
    h = jnp.maximum(dinv * (a0_ref[...] + a1_ref[...] + y_ref[...]) + b_ref[...], 0.0)
    y2_ref[...] = dinv * jnp.dot(h, w_ref[...], preferred_element_type=F32)


def _k3_body(a0_ref, a1_ref, y_ref, dinv_ref, b_ref, wa_ref, wm_ref,
             scal_ref, batch_ref, wo_ref, bo_ref, out_ref):
    dinv = dinv_ref[...]
    h = jnp.maximum(dinv * (a0_ref[...] + a1_ref[...] + y_ref[...]) + b_ref[...], 0.0)
    sa = jnp.sum(h * wa_ref[...], axis=1, keepdims=True) + scal_ref[0, 0]
    sm = jnp.sum(h * wm_ref[...], axis=1, keepdims=True) + scal_ref[0, 1]
    z = h * (sa * jax.nn.sigmoid(sm))
    seg = lax.broadcasted_iota(jnp.int32, (B, N), 0)
    onehot = (batch_ref[...] == seg).astype(F32)
    pooled = jnp.dot(onehot, z, preferred_element_type=F32)
    out_ref[...] = jnp.dot(pooled, wo_ref[...], preferred_element_type=F32) + bo_ref[...]


def kernel(x, edge_index, batch, W1, b1, W2, b2, Wa, ba, Wm, bm, Wo, bo):
    src = edge_index[0]
    dst = edge_index[1]
    pad = ((0, 0), (0, SLAB - EW))
    srcp = jnp.pad(src.reshape(NW, EW), pad).reshape(NW * NCHUNK, C)
    dstp = jnp.pad(dst.reshape(NW, EW), pad, constant_values=TRASH)
    dstp = dstp.reshape(NW * NCHUNK, C)

    degp = _degree(dstp)
    da = degp[0, :N, :]
    db = degp[1, :N, :]

    y1, dinv = pl.pallas_call(
        _k1_body,
        out_shape=(jax.ShapeDtypeStruct((N, 128), F32),
                   jax.ShapeDtypeStruct((N, 128), F32)),
    )(x, W1, da, db)

    acc1 = _edge_scatter(y1, srcp, dstp)

    y2 = pl.pallas_call(
        _k2_body,
        out_shape=jax.ShapeDtypeStruct((N, 128), F32),
    )(acc1[0, :N, :], acc1[1, :N, :], y1, dinv, b1.reshape(1, 128), W2)

    acc2 = _edge_scatter(y2, srcp, dstp)

    scal = jnp.stack([ba, bm], axis=1).astype(F32)
    out = pl.pallas_call(
        _k3_body,
        out_shape=jax.ShapeDtypeStruct((B, OUT), F32),
    )(acc2[0, :N, :], acc2[1, :N, :], y2, dinv, b2.reshape(1, 128),
      Wa.reshape(1, 128), Wm.reshape(1, 128), scal, batch.reshape(1, N),
      Wo, bo.reshape(1, OUT))
    return out


# speedup vs baseline: 7.2690x; 7.2690x over previous
import functools

import jax
import jax.numpy as jnp
from jax import lax
from jax.experimental import pallas as pl
from jax.experimental.pallas import tpu as pltpu
from jax.experimental.pallas import tpu_sc as plsc

N = 10000
E = 320000
D = 128
B = 64
OUT = 10

NC = 2
NS = 16
NW = NC * NS
EW = E // NW
C = 128
NCHUNK = 80
SLAB = NCHUNK * C
NACC = 10240
TRASH = N
RPT = NACC // NS
F32 = jnp.float32

_mesh = plsc.VectorSubcoreMesh(core_axis_name="c", subcore_axis_name="s")


def _zero_buf(buf, rows, width):
    z = jnp.zeros((16,), F32)

    def row(r, carry):
        for k in range(width // 16):
            buf[r, pl.ds(k * 16, 16)] = z
        return carry

    lax.fori_loop(0, rows, row, 0)


@functools.partial(
    pl.kernel,
    out_type=jax.ShapeDtypeStruct((NC, NACC, 128), F32),
    mesh=_mesh,
    scratch_types=[
        pltpu.VMEM((NCHUNK, C), jnp.int32),
        pltpu.VMEM((NCHUNK, C), jnp.int32),
        pltpu.VMEM((C, 128), F32),
        pltpu.VMEM_SHARED((NACC, 128), F32),
        pltpu.SemaphoreType.DMA,
    ],
)
def _edge_scatter(y, srch, dsth, out, srcv, dstv, buf, acc, sem):
    c = lax.axis_index("c")
    s = lax.axis_index("s")
    wid = c * NS + s

    pltpu.sync_copy(srch.at[pl.ds(wid * NCHUNK, NCHUNK)], srcv)
    pltpu.sync_copy(dsth.at[pl.ds(wid * NCHUNK, NCHUNK)], dstv)

    _zero_buf(buf, 128, 128)
    base = s * RPT
    for off in range(0, RPT, 128):
        pltpu.sync_copy(buf, acc.at[pl.ds(base + off, 128)])
    plsc.subcore_barrier()

    def step(j, carry):
        pltpu.async_copy(y.at[srcv.at[j]], buf, sem).wait()
        pltpu.sync_copy(buf, acc.at[dstv.at[j]], add=True)
        return carry

    lax.fori_loop(0, NCHUNK, step, 0)
    plsc.subcore_barrier()

    for off in range(0, RPT, 128):
        pltpu.sync_copy(acc.at[pl.ds(base + off, 128)], buf)
        pltpu.sync_copy(buf, out.at[c, pl.ds(base + off, 128)])


@functools.partial(
    pl.kernel,
    out_type=jax.ShapeDtypeStruct((NC, NACC, 128), F32),
    mesh=_mesh,
    scratch_types=[
        pltpu.VMEM((NCHUNK, C), jnp.int32),
        pltpu.VMEM((C, 16), F32),
        pltpu.VMEM((C, 16), F32),
        pltpu.VMEM((C, 128), F32),
        pltpu.VMEM_SHARED((NACC, 16), F32),
    ],
)
def _degree(dsth, out, dstv, ones, dbuf, bbuf, acc):
    c = lax.axis_index("c")
    s = lax.axis_index("s")
    wid = c * NS + s

    pltpu.sync_copy(dsth.at[pl.ds(wid * NCHUNK, NCHUNK)], dstv)

    one = jnp.ones((16,), F32)

    def fill(r, carry):
        ones[r, pl.ds(0, 16)] = one
        return carry

    lax.fori_loop(0, C, fill, 0)
    _zero_buf(dbuf, C, 16)

    base = s * RPT
    for off in range(0, RPT, 128):
        pltpu.sync_copy(dbuf, acc.at[pl.ds(base + off, 128)])
    plsc.subcore_barrier()

    def step(j, carry):
        pltpu.sync_copy(ones, acc.at[dstv.at[j]], add=True)
        return carry

    lax.fori_loop(0, NCHUNK, step, 0)
    plsc.subcore_barrier()

    for off in range(0, RPT, 128):
        pltpu.sync_copy(acc.at[pl.ds(base + off, 128)], dbuf)

        def brow(r, carry):
            v = dbuf[r, pl.ds(0, 16)]
            for k in range(8):
                bbuf[r, pl.ds(k * 16, 16)] = v
            return carry

        lax.fori_loop(0, C, brow, 0)
        pltpu.sync_copy(bbuf, out.at[c, pl.ds(base + off, 128)])


def _k1_body(x_ref, w_ref, da_ref, db_ref, y_ref, dinv_ref):
    deg = da_ref[...] + db_ref[...] + 1.0
    dinv = lax.rsqrt(deg)
    xw = jnp.dot(x_ref[...], w_ref[...], preferred_element_type=F32)
    y_ref[...] = dinv * xw
    dinv_ref[...] = dinv


def _k2_body(a0_ref, a1_ref, y_ref, dinv_ref, b_ref, w_ref, y2_ref):
    dinv = dinv_ref[...]
    h = jnp.maximum(dinv * (a0_ref[...] + a1_ref[...] + y_ref[...]) + b_ref[...], 0.0)
    y2_ref[...] = dinv * jnp.dot(h, w_ref[...], preferred_element_type=F32)


def _k3_body(a0_ref, a1_ref, y_ref, dinv_ref, b_ref, wa_ref, wm_ref,
             scal_ref, batch_ref, wo_ref, bo_ref, out_ref):
    dinv = dinv_ref[...]
    h = jnp.maximum(dinv * (a0_ref[...] + a1_ref[...] + y_ref[...]) + b_ref[...], 0.0)
    sa = jnp.sum(h * wa_ref[...], axis=1, keepdims=True) + scal_ref[0, 0]
    sm = jnp.sum(h * wm_ref[...], axis=1, keepdims=True) + scal_ref[0, 1]
    z = h * (sa * jax.nn.sigmoid(sm))
    seg = lax.broadcasted_iota(jnp.int32, (B, N), 0)
    onehot = (batch_ref[...] == seg).astype(F32)
    pooled = jnp.dot(onehot, z, preferred_element_type=F32)
    out_ref[...] = jnp.dot(pooled, wo_ref[...], preferred_element_type=F32) + bo_ref[...]


def kernel(x, edge_index, batch, W1, b1, W2, b2, Wa, ba, Wm, bm, Wo, bo):
    src = edge_index[0]
    dst = edge_index[1]
    pad = ((0, 0), (0, SLAB - EW))
    srcp = jnp.pad(src.reshape(NW, EW), pad).reshape(NW * NCHUNK, C)
    dstp = jnp.pad(dst.reshape(NW, EW), pad, constant_values=TRASH)
    dstp = dstp.reshape(NW * NCHUNK, C)

    degj = jnp.zeros((N,), F32).at[dst].add(1.0)
    da = jnp.broadcast_to(degj[:, None], (N, 128))
    db = jnp.zeros((N, 128), F32)

    y1, dinv = pl.pallas_call(
        _k1_body,
        out_shape=(jax.ShapeDtypeStruct((N, 128), F32),
                   jax.ShapeDtypeStruct((N, 128), F32)),
    )(x, W1, da, db)

    acc1 = _edge_scatter(y1, srcp, dstp)

    y2 = pl.pallas_call(
        _k2_body,
        out_shape=jax.ShapeDtypeStruct((N, 128), F32),
    )(acc1[0, :N, :], acc1[1, :N, :], y1, dinv, b1.reshape(1, 128), W2)

    acc2 = _edge_scatter(y2, srcp, dstp)

    scal = jnp.stack([ba, bm], axis=1).astype(F32)
    out = pl.pallas_call(
        _k3_body,
        out_shape=jax.ShapeDtypeStruct((B, OUT), F32),
    )(acc2[0, :N, :], acc2[1, :N, :], y2, dinv, b2.reshape(1, 128),
      Wa.reshape(1, 128), Wm.reshape(1, 128), scal, batch.reshape(1, N),
      Wo, bo.reshape(1, OUT))
    return out


# speedup vs baseline: 9.3871x; 1.2914x over previous
import functools

import jax
import jax.numpy as jnp
from jax import lax
from jax.experimental import pallas as pl
from jax.experimental.pallas import tpu as pltpu
from jax.experimental.pallas import tpu_sc as plsc

N = 10000
E = 320000
D = 128
B = 64
OUT = 10

NC = 2
NS = 16
NW = NC * NS
EW = E // NW
C = 128
NCHUNK = 80
SLAB = NCHUNK * C
NACC = 10240
TRASH = N
RPT = NACC // NS
F32 = jnp.float32

_mesh = plsc.VectorSubcoreMesh(core_axis_name="c", subcore_axis_name="s")


def _zero_buf(buf, rows, width):
    z = jnp.zeros((16,), F32)

    def row(r, carry):
        for k in range(width // 16):
            buf[r, pl.ds(k * 16, 16)] = z
        return carry

    lax.fori_loop(0, rows, row, 0)


@functools.partial(
    pl.kernel,
    out_type=jax.ShapeDtypeStruct((NC, NACC, 128), F32),
    mesh=_mesh,
    scratch_types=[
        pltpu.VMEM((NCHUNK, C), jnp.int32),
        pltpu.VMEM((NCHUNK, C), jnp.int32),
        pltpu.VMEM((C, 128), F32),
        pltpu.VMEM_SHARED((NACC, 128), F32),
        pltpu.SemaphoreType.DMA,
    ],
)
def _edge_scatter(y, srch, dsth, out, srcv, dstv, buf, acc, sem):
    c = lax.axis_index("c")
    s = lax.axis_index("s")
    wid = c * NS + s

    pltpu.sync_copy(srch.at[pl.ds(wid * NCHUNK, NCHUNK)], srcv)
    pltpu.sync_copy(dsth.at[pl.ds(wid * NCHUNK, NCHUNK)], dstv)

    _zero_buf(buf, 128, 128)
    base = s * RPT
    for off in range(0, RPT, 128):
        pltpu.sync_copy(buf, acc.at[pl.ds(base + off, 128)])
    plsc.subcore_barrier()

    def step(j, carry):
        pltpu.async_copy(y.at[srcv.at[j]], buf, sem).wait()
        pltpu.sync_copy(buf, acc.at[dstv.at[j]], add=True)
        return carry

    lax.fori_loop(0, NCHUNK, step, 0)
    plsc.subcore_barrier()

    for off in range(0, RPT, 128):
        pltpu.sync_copy(acc.at[pl.ds(base + off, 128)], buf)
        pltpu.sync_copy(buf, out.at[c, pl.ds(base + off, 128)])


@functools.partial(
    pl.kernel,
    out_type=jax.ShapeDtypeStruct((NC, NACC, 128), F32),
    mesh=_mesh,
    scratch_types=[
        pltpu.VMEM((NCHUNK, C), jnp.int32),
        pltpu.VMEM((C, 128), F32),
        pltpu.VMEM_SHARED((NACC, 128), F32),
    ],
)
def _degree(dsth, out, dstv, buf, acc):
    c = lax.axis_index("c")
    s = lax.axis_index("s")
    wid = c * NS + s

    pltpu.sync_copy(dsth.at[pl.ds(wid * NCHUNK, NCHUNK)], dstv)

    _zero_buf(buf, C, 128)
    base = s * RPT
    for off in range(0, RPT, 128):
        pltpu.sync_copy(buf, acc.at[pl.ds(base + off, 128)])
    plsc.subcore_barrier()

    one = jnp.ones((16,), F32)

    def orow(r, carry):
        for k in range(8):
            buf[r, pl.ds(k * 16, 16)] = one
        return carry

    lax.fori_loop(0, C, orow, 0)

    def step(j, carry):
        pltpu.sync_copy(buf, acc.at[dstv.at[j]], add=True)
        return carry

    lax.fori_loop(0, NCHUNK, step, 0)
    plsc.subcore_barrier()

    for off in range(0, RPT, 128):
        pltpu.sync_copy(acc.at[pl.ds(base + off, 128)], buf)
        pltpu.sync_copy(buf, out.at[c, pl.ds(base + off, 128)])


def _k1_body(x_ref, w_ref, da_ref, db_ref, y_ref, dinv_ref):
    deg = da_ref[...] + db_ref[...] + 1.0
    dinv = lax.rsqrt(deg)
    xw = jnp.dot(x_ref[...], w_ref[...], preferred_element_type=F32)
    y_ref[...] = dinv * xw
    dinv_ref[...] = dinv


def _k2_body(a0_ref, a1_ref, y_ref, dinv_ref, b_ref, w_ref, y2_ref):
    dinv = dinv_ref[...]
    h = jnp.maximum(dinv * (a0_ref[...] + a1_ref[...] + y_ref[...]) + b_ref[...], 0.0)
    y2_ref[...] = dinv * jnp.dot(h, w_ref[...], preferred_element_type=F32)


def _k3_body(a0_ref, a1_ref, y_ref, dinv_ref, b_ref, wa_ref, wm_ref,
             scal_ref, batch_ref, wo_ref, bo_ref, out_ref):
    dinv = dinv_ref[...]
    h = jnp.maximum(dinv * (a0_ref[...] + a1_ref[...] + y_ref[...]) + b_ref[...], 0.0)
    sa = jnp.sum(h * wa_ref[...], axis=1, keepdims=True) + scal_ref[0, 0]
    sm = jnp.sum(h * wm_ref[...], axis=1, keepdims=True) + scal_ref[0, 1]
    z = h * (sa * jax.nn.sigmoid(sm))
    seg = lax.broadcasted_iota(jnp.int32, (B, N), 0)
    onehot = (batch_ref[...] == seg).astype(F32)
    pooled = jnp.dot(onehot, z, preferred_element_type=F32)
    out_ref[...] = jnp.dot(pooled, wo_ref[...], preferred_element_type=F32) + bo_ref[...]


def kernel(x, edge_index, batch, W1, b1, W2, b2, Wa, ba, Wm, bm, Wo, bo):
    src = edge_index[0]
    dst = edge_index[1]
    pad = ((0, 0), (0, SLAB - EW))
    srcp = jnp.pad(src.reshape(NW, EW), pad).reshape(NW * NCHUNK, C)
    dstp = jnp.pad(dst.reshape(NW, EW), pad, constant_values=TRASH)
    dstp = dstp.reshape(NW * NCHUNK, C)

    degp = _degree(dstp)
    da = degp[0, :N, :]
    db = degp[1, :N, :]

    y1, dinv = pl.pallas_call(
        _k1_body,
        out_shape=(jax.ShapeDtypeStruct((N, 128), F32),
                   jax.ShapeDtypeStruct((N, 128), F32)),
    )(x, W1, da, db)

    acc1 = _edge_scatter(y1, srcp, dstp)

    y2 = pl.pallas_call(
        _k2_body,
        out_shape=jax.ShapeDtypeStruct((N, 128), F32),
    )(acc1[0, :N, :], acc1[1, :N, :], y1, dinv, b1.reshape(1, 128), W2)

    acc2 = _edge_scatter(y2, srcp, dstp)

    scal = jnp.stack([ba, bm], axis=1).astype(F32)
    out = pl.pallas_call(
        _k3_body,
        out_shape=jax.ShapeDtypeStruct((B, OUT), F32),
    )(acc2[0, :N, :], acc2[1, :N, :], y2, dinv, b2.reshape(1, 128),
      Wa.reshape(1, 128), Wm.reshape(1, 128), scal, batch.reshape(1, N),
      Wo, bo.reshape(1, OUT))
    return out


# speedup vs baseline: 10.6227x; 1.1316x over previous
import functools

import jax
import jax.numpy as jnp
from jax import lax
from jax.experimental import pallas as pl
from jax.experimental.pallas import tpu as pltpu
from jax.experimental.pallas import tpu_sc as plsc

N = 10000
E = 320000
D = 128
B = 64
OUT = 10

NC = 2
NS = 16
NW = NC * NS
EW = E // NW
C = 128
NCHUNK = 80
SLAB = NCHUNK * C
NACC = 10240
TRASH = N
RPT = NACC // NS
F32 = jnp.float32

_mesh = plsc.VectorSubcoreMesh(core_axis_name="c", subcore_axis_name="s")


def _zero_buf(buf, rows, width):
    z = jnp.zeros((16,), F32)

    def row(r, carry):
        for k in range(width // 16):
            buf[r, pl.ds(k * 16, 16)] = z
        return carry

    lax.fori_loop(0, rows, row, 0)


@functools.partial(
    pl.kernel,
    out_type=jax.ShapeDtypeStruct((NC, NACC, 128), F32),
    mesh=_mesh,
    scratch_types=[
        pltpu.VMEM((NCHUNK // 2, C), jnp.int32),
        pltpu.VMEM((NCHUNK // 2, C), jnp.int32),
        pltpu.VMEM((C, 128), F32),
        pltpu.VMEM((C, 128), F32),
        pltpu.VMEM_SHARED((NACC, 128), F32),
        pltpu.SemaphoreType.DMA,
        pltpu.SemaphoreType.DMA,
    ],
)
def _edge_scatter(y, srch, dsth, out, srcv, dstv, b0, b1, acc, s0, s1):
    c = lax.axis_index("c")
    s = lax.axis_index("s")
    wid = c * NS + s
    bufs = (b0, b1)
    sems = (s0, s1)
    nbuf = 2
    half = NCHUNK // 2

    _zero_buf(b0, 128, 128)
    base = s * RPT
    for off in range(0, RPT, 128):
        pltpu.sync_copy(b0, acc.at[pl.ds(base + off, 128)])
    plsc.subcore_barrier()

    for h in range(2):
        pltpu.sync_copy(srch.at[pl.ds((wid * 2 + h) * half, half)], srcv)
        pltpu.sync_copy(dsth.at[pl.ds((wid * 2 + h) * half, half)], dstv)
        for b in range(nbuf):
            pltpu.async_copy(y.at[srcv.at[b]], bufs[b], sems[b])

        def group(g, carry):
            for b in range(nbuf):
                j = g * nbuf + b
                pltpu.make_async_copy(y.at[srcv.at[j]], bufs[b], sems[b]).wait()
                pltpu.sync_copy(bufs[b], acc.at[dstv.at[j]], add=True)

                @pl.when(g < half // nbuf - 1)
                def _():
                    pltpu.async_copy(y.at[srcv.at[j + nbuf]], bufs[b], sems[b])
            return carry

        lax.fori_loop(0, half // nbuf, group, 0)
    plsc.subcore_barrier()

    for off in range(0, RPT, 128):
        pltpu.sync_copy(acc.at[pl.ds(base + off, 128)], b0)
        pltpu.sync_copy(b0, out.at[c, pl.ds(base + off, 128)])


@functools.partial(
    pl.kernel,
    out_type=jax.ShapeDtypeStruct((NC, NACC, 128), F32),
    mesh=_mesh,
    scratch_types=[
        pltpu.VMEM((NCHUNK, C), jnp.int32),
        pltpu.VMEM((C, 128), F32),
        pltpu.VMEM_SHARED((NACC, 128), F32),
    ],
)
def _degree(dsth, out, dstv, buf, acc):
    c = lax.axis_index("c")
    s = lax.axis_index("s")
    wid = c * NS + s

    pltpu.sync_copy(dsth.at[pl.ds(wid * NCHUNK, NCHUNK)], dstv)

    _zero_buf(buf, C, 128)
    base = s * RPT
    for off in range(0, RPT, 128):
        pltpu.sync_copy(buf, acc.at[pl.ds(base + off, 128)])
    plsc.subcore_barrier()

    one = jnp.ones((16,), F32)

    def orow(r, carry):
        for k in range(8):
            buf[r, pl.ds(k * 16, 16)] = one
        return carry

    lax.fori_loop(0, C, orow, 0)

    def step(j, carry):
        pltpu.sync_copy(buf, acc.at[dstv.at[j]], add=True)
        return carry

    lax.fori_loop(0, NCHUNK, step, 0)
    plsc.subcore_barrier()

    for off in range(0, RPT, 128):
        pltpu.sync_copy(acc.at[pl.ds(base + off, 128)], buf)
        pltpu.sync_copy(buf, out.at[c, pl.ds(base + off, 128)])


def _k1_body(x_ref, w_ref, da_ref, db_ref, y_ref, dinv_ref):
    deg = da_ref[...] + db_ref[...] + 1.0
    dinv = lax.rsqrt(deg)
    xw = jnp.dot(x_ref[...], w_ref[...], preferred_element_type=F32)
    y_ref[...] = dinv * xw
    dinv_ref[...] = dinv


def _k2_body(a0_ref, a1_ref, y_ref, dinv_ref, b_ref, w_ref, y2_ref):
    dinv = dinv_ref[...]
    h = jnp.maximum(dinv * (a0_ref[...] + a1_ref[...] + y_ref[...]) + b_ref[...], 0.0)
    y2_ref[...] = dinv * jnp.dot(h, w_ref[...], preferred_element_type=F32)


def _k3_body(a0_ref, a1_ref, y_ref, dinv_ref, b_ref, wa_ref, wm_ref,
             scal_ref, batch_ref, wo_ref, bo_ref, out_ref):
    dinv = dinv_ref[...]
    h = jnp.maximum(dinv * (a0_ref[...] + a1_ref[...] + y_ref[...]) + b_ref[...], 0.0)
    sa = jnp.sum(h * wa_ref[...], axis=1, keepdims=True) + scal_ref[0, 0]
    sm = jnp.sum(h * wm_ref[...], axis=1, keepdims=True) + scal_ref[0, 1]
    z = h * (sa * jax.nn.sigmoid(sm))
    seg = lax.broadcasted_iota(jnp.int32, (B, N), 0)
    onehot = (batch_ref[...] == seg).astype(F32)
    pooled = jnp.dot(onehot, z, preferred_element_type=F32)
    out_ref[...] = jnp.dot(pooled, wo_ref[...], preferred_element_type=F32) + bo_ref[...]


def kernel(x, edge_index, batch, W1, b1, W2, b2, Wa, ba, Wm, bm, Wo, bo):
    src = edge_index[0]
    dst = edge_index[1]
    pad = ((0, 0), (0, SLAB - EW))
    srcp = jnp.pad(src.reshape(NW, EW), pad).reshape(NW * NCHUNK, C)
    dstp = jnp.pad(dst.reshape(NW, EW), pad, constant_values=TRASH)
    dstp = dstp.reshape(NW * NCHUNK, C)

    degp = _degree(dstp)
    da = degp[0, :N, :]
    db = degp[1, :N, :]

    y1, dinv = pl.pallas_call(
        _k1_body,
        out_shape=(jax.ShapeDtypeStruct((N, 128), F32),
                   jax.ShapeDtypeStruct((N, 128), F32)),
    )(x, W1, da, db)

    acc1 = _edge_scatter(y1, srcp, dstp)

    y2 = pl.pallas_call(
        _k2_body,
        out_shape=jax.ShapeDtypeStruct((N, 128), F32),
    )(acc1[0, :N, :], acc1[1, :N, :], y1, dinv, b1.reshape(1, 128), W2)

    acc2 = _edge_scatter(y2, srcp, dstp)

    scal = jnp.stack([ba, bm], axis=1).astype(F32)
    out = pl.pallas_call(
        _k3_body,
        out_shape=jax.ShapeDtypeStruct((B, OUT), F32),
    )(acc2[0, :N, :], acc2[1, :N, :], y2, dinv, b2.reshape(1, 128),
      Wa.reshape(1, 128), Wm.reshape(1, 128), scal, batch.reshape(1, N),
      Wo, bo.reshape(1, OUT))
    return out
